# baseline (speedup 1.0000x reference)
import functools

import jax
import jax.numpy as jnp
from jax import lax
from jax.experimental import pallas as pl
from jax.experimental.pallas import tpu as pltpu
from jax.experimental.pallas import tpu_sc as plsc

B = 16384
D = 64
P = 11
A = 4
LANES = 16

NC = 2
NS = 16
NW = NC * NS
ROWS_W = B // NW
G2 = 32
EPS = 1e-8

GROUPS = ((0, 1, 2, 3), (4, 5, 6, 7, 8), (9,), (10,))


def _rsqrt(x):
    i = lax.bitcast_convert_type(x, jnp.int32)
    i = jnp.int32(0x5F3759DF) - (i >> 1)
    y = lax.bitcast_convert_type(i, jnp.float32)
    for _ in range(3):
        y = y * (1.5 - 0.5 * x * y * y)
    return y


def _maxtree(vals):
    vals = list(vals)
    while len(vals) > 1:
        vals = [jnp.maximum(vals[i], vals[i + 1]) for i in range(0, len(vals) - 1, 2)] + (
            [vals[-1]] if len(vals) % 2 else [])
    return vals[0]


def _sc_body(et_hbm, av_hbm, wb_hbm, out_hbm, et_v, out_v, av_v, wb_v,
             sem_e, sem_a):
    wid = lax.axis_index("s") * NC + lax.axis_index("c")

    cp_e = pltpu.async_copy(
        et_hbm.at[:, pl.ds(wid * (ROWS_W // 128), ROWS_W // 128)], et_v, sem_e)
    cp_a = pltpu.async_copy(av_hbm, av_v, sem_a)
    pltpu.sync_copy(wb_hbm, wb_v)
    cp_a.wait()

    def nb_body(d, accs):
        return tuple(accs[p] + av_v[d * P + p, :] * av_v[d * P + p, :]
                     for p in range(P))
    nb2s = lax.fori_loop(0, D, nb_body,
                         tuple(jnp.zeros((LANES,), jnp.float32) for _ in range(P)))
    inv_nb = [_rsqrt(jnp.maximum(nb2, EPS * EPS)) for nb2 in nb2s]
    cp_e.wait()

    def finish(acc, n2, lr0):
        ca = _rsqrt(jnp.maximum(n2, EPS * EPS))
        sims = [acc[p] * inv_nb[p] for p in range(P)]
        pooled = [_maxtree([sims[p] for p in g]) * ca for g in GROUPS]
        logits = []
        for j in range(A):
            lj = pooled[0] * wb_v[4 * j + 0, :]
            for k in range(1, A):
                lj = lj + pooled[k] * wb_v[4 * j + k, :]
            logits.append(lj + wb_v[A * A + j, :])
        m = _maxtree(logits)
        es = [jnp.exp(l - m) for l in logits]
        tot = (es[0] + es[1]) + (es[2] + es[3])
        rs = _rsqrt(tot)
        inv = rs * rs
        rhi4 = (lr0 >> 7) * 4
        cl = lr0 & 127
        for j in range(A):
            out_v[rhi4 + j, pl.ds(cl, LANES)] = es[j] * inv

    def blk_body(k, carry):
        lr0 = k * G2
        rhi = lr0 >> 7
        rlo = lr0 & 127
        def d_body(dhi, accs):
            acc_a, acc_b, n2a, n2b = list(accs[0]), list(accs[1]), accs[2], accs[3]
            for dlo in range(8):
                va = et_v[dhi, rhi, dlo, pl.ds(rlo, LANES)]
                vb = et_v[dhi, rhi, dlo, pl.ds(rlo + LANES, LANES)]
                row = (dhi * 8 + dlo) * P
                for p in range(P):
                    s = av_v[row + p, :]
                    acc_a[p] = acc_a[p] + va * s
                    acc_b[p] = acc_b[p] + vb * s
                n2a = n2a + va * va
                n2b = n2b + vb * vb
            return tuple(acc_a), tuple(acc_b), n2a, n2b

        z = tuple(jnp.zeros((LANES,), jnp.float32) for _ in range(P))
        zn = jnp.zeros((LANES,), jnp.float32)
        acc_a, acc_b, n2a, n2b = lax.fori_loop(0, D // 8, d_body, (z, z, zn, zn))
        finish(list(acc_a), n2a, lr0)
        finish(list(acc_b), n2b, lr0 + LANES)
        return carry

    lax.fori_loop(0, ROWS_W // G2, blk_body, 0)
    pltpu.sync_copy(out_v, out_hbm.at[pl.ds(wid * (ROWS_W // 128) * A, (ROWS_W // 128) * A)])


@functools.partial(
    pl.kernel,
    out_type=jax.ShapeDtypeStruct((B // 128 * A, 128), jnp.float32),
    mesh=plsc.VectorSubcoreMesh(core_axis_name="c", subcore_axis_name="s"),
    compiler_params=pltpu.CompilerParams(needs_layout_passes=False,
                                         use_tc_tiling_on_sc=False),
    scratch_types=[
        pltpu.VMEM((D // 8, ROWS_W // 128, 8, 128), jnp.float32),
        pltpu.VMEM((ROWS_W // 128 * A, 128), jnp.float32),
        pltpu.VMEM((D * P, LANES), jnp.float32),
        pltpu.VMEM((A * A + A, LANES), jnp.float32),
        pltpu.SemaphoreType.DMA,
        pltpu.SemaphoreType.DMA,
    ],
)
def _sc_decoder(et_hbm, av_hbm, wb_hbm, out_hbm, et_v, out_v, av_v, wb_v,
                sem_e, sem_a):
    _sc_body(et_hbm, av_hbm, wb_hbm, out_hbm, et_v, out_v, av_v, wb_v,
             sem_e, sem_a)


def kernel(embedded_words, action_vectors, W, b):
    av = action_vectors.reshape(D * P).astype(jnp.float32)
    av_splat = jnp.broadcast_to(av[:, None], (D * P, LANES))
    wb = jnp.concatenate([W.reshape(A * A), b.reshape(A)]).astype(jnp.float32)
    wb_splat = jnp.broadcast_to(wb[:, None], (A * A + A, LANES))
    et4 = embedded_words.reshape(128, 128, 8, 8).transpose(2, 0, 3, 1)
    o2 = _sc_decoder(et4, av_splat, wb_splat)
    return o2.reshape(B // 128, A, 128).transpose(0, 2, 1).reshape(B, A)


# speedup vs baseline: 1.3596x; 1.3596x over previous
import functools

import jax
import jax.numpy as jnp
from jax import lax
from jax.experimental import pallas as pl
from jax.experimental.pallas import tpu as pltpu
from jax.experimental.pallas import tpu_sc as plsc

B = 16384
D = 64
P = 11
A = 4
LANES = 16

SC_B = 8192
TC_B = B - SC_B
TCBLK = 512

NC = 2
NS = 16
NW = NC * NS
ROWS_W = SC_B // NW
G2 = 32
EPS = 1e-8

GROUPS = ((0, 1, 2, 3), (4, 5, 6, 7, 8), (9,), (10,))


def _rsqrt(x):
    i = lax.bitcast_convert_type(x, jnp.int32)
    i = jnp.int32(0x5F3759DF) - (i >> 1)
    y = lax.bitcast_convert_type(i, jnp.float32)
    for _ in range(3):
        y = y * (1.5 - 0.5 * x * y * y)
    return y


def _maxtree(vals):
    vals = list(vals)
    while len(vals) > 1:
        vals = [jnp.maximum(vals[i], vals[i + 1]) for i in range(0, len(vals) - 1, 2)] + (
            [vals[-1]] if len(vals) % 2 else [])
    return vals[0]


def _sc_body(et_hbm, av_hbm, wb_hbm, out_hbm, et_v, out_v, av_v, wb_v,
             sem_e, sem_a):
    wid = lax.axis_index("s") * NC + lax.axis_index("c")

    cp_e = pltpu.async_copy(
        et_hbm.at[:, pl.ds(wid * (ROWS_W // 128), ROWS_W // 128)], et_v, sem_e)
    cp_a = pltpu.async_copy(av_hbm, av_v, sem_a)
    pltpu.sync_copy(wb_hbm, wb_v)
    cp_a.wait()

    inv_nb = []
    for p in range(P):
        nb2 = jnp.zeros((LANES,), jnp.float32)
        for d in range(D):
            s = av_v[d * P + p, :]
            nb2 = nb2 + s * s
        inv_nb.append(_rsqrt(jnp.maximum(nb2, EPS * EPS)))
    cp_e.wait()

    def finish(acc, n2, lr0):
        ca = _rsqrt(jnp.maximum(n2, EPS * EPS))
        sims = [acc[p] * inv_nb[p] for p in range(P)]
        pooled = [_maxtree([sims[p] for p in g]) * ca for g in GROUPS]
        logits = []
        for j in range(A):
            lj = pooled[0] * wb_v[4 * j + 0, :]
            for k in range(1, A):
                lj = lj + pooled[k] * wb_v[4 * j + k, :]
            logits.append(lj + wb_v[A * A + j, :])
        m = _maxtree(logits)
        es = [jnp.exp(l - m) for l in logits]
        tot = (es[0] + es[1]) + (es[2] + es[3])
        rs = _rsqrt(tot)
        inv = rs * rs
        rhi4 = (lr0 >> 7) * 4
        cl = lr0 & 127
        for j in range(A):
            out_v[rhi4 + j, pl.ds(cl, LANES)] = es[j] * inv

    def blk_body(k, carry):
        lr0 = k * G2
        rhi = lr0 >> 7
        rlo = lr0 & 127
        acc_a = [jnp.zeros((LANES,), jnp.float32) for _ in range(P)]
        acc_b = [jnp.zeros((LANES,), jnp.float32) for _ in range(P)]
        n2a = jnp.zeros((LANES,), jnp.float32)
        n2b = jnp.zeros((LANES,), jnp.float32)
        for d in range(D):
            va = et_v[d >> 3, rhi, d & 7, pl.ds(rlo, LANES)]
            vb = et_v[d >> 3, rhi, d & 7, pl.ds(rlo + LANES, LANES)]
            for p in range(P):
                s = av_v[d * P + p, :]
                acc_a[p] = acc_a[p] + va * s
                acc_b[p] = acc_b[p] + vb * s
            n2a = n2a + va * va
            n2b = n2b + vb * vb
        finish(acc_a, n2a, lr0)
        finish(acc_b, n2b, lr0 + LANES)
        return carry

    lax.fori_loop(0, ROWS_W // G2, blk_body, 0)
    pltpu.sync_copy(out_v, out_hbm.at[pl.ds(wid * (ROWS_W // 128) * A, (ROWS_W // 128) * A)])


@functools.partial(
    pl.kernel,
    out_type=jax.ShapeDtypeStruct((SC_B // 128 * A, 128), jnp.float32),
    mesh=plsc.VectorSubcoreMesh(core_axis_name="c", subcore_axis_name="s"),
    compiler_params=pltpu.CompilerParams(needs_layout_passes=False,
                                         use_tc_tiling_on_sc=False),
    scratch_types=[
        pltpu.VMEM((D // 8, ROWS_W // 128, 8, 128), jnp.float32),
        pltpu.VMEM((ROWS_W // 128 * A, 128), jnp.float32),
        pltpu.VMEM((D * P, LANES), jnp.float32),
        pltpu.VMEM((A * A + A, LANES), jnp.float32),
        pltpu.SemaphoreType.DMA,
        pltpu.SemaphoreType.DMA,
    ],
)
def _sc_decoder(et_hbm, av_hbm, wb_hbm, out_hbm, et_v, out_v, av_v, wb_v,
                sem_e, sem_a):
    _sc_body(et_hbm, av_hbm, wb_hbm, out_hbm, et_v, out_v, av_v, wb_v,
             sem_e, sem_a)


def _tc_body(et_ref, av_ref, wb_ref, out_ref):
    e = et_ref[...]
    av = av_ref[...]
    nb2 = jnp.sum(av * av, axis=0, keepdims=True)
    inv_nb = 1.0 / jnp.maximum(jnp.sqrt(nb2), EPS)
    raw = jnp.dot(av.T, e, preferred_element_type=jnp.float32)
    n2 = jnp.sum(e * e, axis=0, keepdims=True)
    ca = 1.0 / jnp.maximum(jnp.sqrt(n2), EPS)
    sims = raw * inv_nb.reshape(16, 1)
    pooled = jnp.concatenate([
        jnp.max(sims[0:4], axis=0, keepdims=True),
        jnp.max(sims[4:9], axis=0, keepdims=True),
        sims[9:10],
        sims[10:11],
    ], axis=0) * ca
    w = wb_ref[0:A, :]
    bias = wb_ref[A:A + 1, :].reshape(A, 1)
    logits = jnp.dot(w, pooled, preferred_element_type=jnp.float32) + bias
    m = jnp.max(logits, axis=0, keepdims=True)
    es = jnp.exp(logits - m)
    probs = es / jnp.sum(es, axis=0, keepdims=True)
    out_ref[...] = probs.reshape(A, TCBLK // 128, 128).transpose(1, 0, 2).reshape(
        TCBLK // 128 * A, 128)


_tc_decoder = pl.pallas_call(
    _tc_body,
    grid=(TC_B // TCBLK,),
    in_specs=[
        pl.BlockSpec((D, TCBLK), lambda i: (0, SC_B // TCBLK + i)),
        pl.BlockSpec((D, 16), lambda i: (0, 0)),
        pl.BlockSpec((2 * A, A), lambda i: (0, 0)),
    ],
    out_specs=pl.BlockSpec((TCBLK // 128 * A, 128), lambda i: (i, 0)),
    out_shape=jax.ShapeDtypeStruct((TC_B // 128 * A, 128), jnp.float32),
)


def kernel(embedded_words, action_vectors, W, b):
    av = action_vectors.reshape(D * P).astype(jnp.float32)
    av_splat = jnp.broadcast_to(av[:, None], (D * P, LANES))
    wb = jnp.concatenate([W.reshape(A * A), b.reshape(A)]).astype(jnp.float32)
    wb_splat = jnp.broadcast_to(wb[:, None], (A * A + A, LANES))
    et4 = embedded_words.reshape(128, 128, 8, 8).transpose(2, 0, 3, 1)
    o_sc = _sc_decoder(et4, av_splat, wb_splat)
    et = embedded_words.T
    av_pad = jnp.zeros((D, 16), jnp.float32).at[:, :P].set(
        action_vectors.reshape(D, P).astype(jnp.float32))
    wb_tc = jnp.zeros((2 * A, A), jnp.float32).at[:A].set(W).at[A].set(b)
    o_tc = _tc_decoder(et, av_pad, wb_tc)
    o2 = jnp.concatenate([o_sc, o_tc], axis=0)
    return o2.reshape(B // 128, A, 128).transpose(0, 2, 1).reshape(B, A)


# speedup vs baseline: 1.3795x; 1.0147x over previous
import functools

import jax
import jax.numpy as jnp
from jax import lax
from jax.experimental import pallas as pl
from jax.experimental.pallas import tpu as pltpu
from jax.experimental.pallas import tpu_sc as plsc

B = 16384
D = 64
P = 11
A = 4
LANES = 16

SC_B = 8192
TC_B = B - SC_B
TCBLK = 2048

NC = 2
NS = 16
NW = NC * NS
ROWS_W = SC_B // NW
G2 = 32
EPS = 1e-8

GROUPS = ((0, 1, 2, 3), (4, 5, 6, 7, 8), (9,), (10,))


def _rsqrt(x):
    i = lax.bitcast_convert_type(x, jnp.int32)
    i = jnp.int32(0x5F3759DF) - (i >> 1)
    y = lax.bitcast_convert_type(i, jnp.float32)
    for _ in range(3):
        y = y * (1.5 - 0.5 * x * y * y)
    return y


def _maxtree(vals):
    vals = list(vals)
    while len(vals) > 1:
        vals = [jnp.maximum(vals[i], vals[i + 1]) for i in range(0, len(vals) - 1, 2)] + (
            [vals[-1]] if len(vals) % 2 else [])
    return vals[0]


def _sc_body(et_hbm, av_hbm, wb_hbm, out_hbm, et_v, out_v, av_v, wb_v,
             sem_e, sem_a):
    wid = lax.axis_index("s") * NC + lax.axis_index("c")

    cp_e = pltpu.async_copy(
        et_hbm.at[:, pl.ds(wid * (ROWS_W // 128), ROWS_W // 128)], et_v, sem_e)
    cp_a = pltpu.async_copy(av_hbm, av_v, sem_a)
    pltpu.sync_copy(wb_hbm, wb_v)
    cp_a.wait()

    inv_nb = []
    for p in range(P):
        nb2 = jnp.zeros((LANES,), jnp.float32)
        for d in range(D):
            s = av_v[d * P + p, :]
            nb2 = nb2 + s * s
        inv_nb.append(_rsqrt(jnp.maximum(nb2, EPS * EPS)))
    cp_e.wait()

    def finish(acc, n2, lr0):
        ca = _rsqrt(jnp.maximum(n2, EPS * EPS))
        sims = [acc[p] * inv_nb[p] for p in range(P)]
        pooled = [_maxtree([sims[p] for p in g]) * ca for g in GROUPS]
        logits = []
        for j in range(A):
            lj = pooled[0] * wb_v[4 * j + 0, :]
            for k in range(1, A):
                lj = lj + pooled[k] * wb_v[4 * j + k, :]
            logits.append(lj + wb_v[A * A + j, :])
        m = _maxtree(logits)
        es = [jnp.exp(l - m) for l in logits]
        tot = (es[0] + es[1]) + (es[2] + es[3])
        rs = _rsqrt(tot)
        inv = rs * rs
        rhi4 = (lr0 >> 7) * 4
        cl = lr0 & 127
        for j in range(A):
            out_v[rhi4 + j, pl.ds(cl, LANES)] = es[j] * inv

    def blk_body(k, carry):
        lr0 = k * G2
        rhi = lr0 >> 7
        rlo = lr0 & 127
        acc_a = [jnp.zeros((LANES,), jnp.float32) for _ in range(P)]
        acc_b = [jnp.zeros((LANES,), jnp.float32) for _ in range(P)]
        n2a = jnp.zeros((LANES,), jnp.float32)
        n2b = jnp.zeros((LANES,), jnp.float32)
        for d in range(D):
            va = et_v[d >> 3, rhi, d & 7, pl.ds(rlo, LANES)]
            vb = et_v[d >> 3, rhi, d & 7, pl.ds(rlo + LANES, LANES)]
            for p in range(P):
                s = av_v[d * P + p, :]
                acc_a[p] = acc_a[p] + va * s
                acc_b[p] = acc_b[p] + vb * s
            n2a = n2a + va * va
            n2b = n2b + vb * vb
        finish(acc_a, n2a, lr0)
        finish(acc_b, n2b, lr0 + LANES)
        return carry

    lax.fori_loop(0, ROWS_W // G2, blk_body, 0)
    pltpu.sync_copy(out_v, out_hbm.at[pl.ds(wid * (ROWS_W // 128) * A, (ROWS_W // 128) * A)])


@functools.partial(
    pl.kernel,
    out_type=jax.ShapeDtypeStruct((SC_B // 128 * A, 128), jnp.float32),
    mesh=plsc.VectorSubcoreMesh(core_axis_name="c", subcore_axis_name="s"),
    compiler_params=pltpu.CompilerParams(needs_layout_passes=False,
                                         use_tc_tiling_on_sc=False),
    scratch_types=[
        pltpu.VMEM((D // 8, ROWS_W // 128, 8, 128), jnp.float32),
        pltpu.VMEM((ROWS_W // 128 * A, 128), jnp.float32),
        pltpu.VMEM((D * P, LANES), jnp.float32),
        pltpu.VMEM((A * A + A, LANES), jnp.float32),
        pltpu.SemaphoreType.DMA,
        pltpu.SemaphoreType.DMA,
    ],
)
def _sc_decoder(et_hbm, av_hbm, wb_hbm, out_hbm, et_v, out_v, av_v, wb_v,
                sem_e, sem_a):
    _sc_body(et_hbm, av_hbm, wb_hbm, out_hbm, et_v, out_v, av_v, wb_v,
             sem_e, sem_a)


def _tc_body(et_ref, av_ref, wb_ref, out_ref):
    e = et_ref[...]
    av = av_ref[...]
    nb2 = jnp.sum(av * av, axis=0, keepdims=True)
    inv_nb = 1.0 / jnp.maximum(jnp.sqrt(nb2), EPS)
    raw = lax.dot_general(av, e, (((0,), (0,)), ((), ())),
                          preferred_element_type=jnp.float32)
    n2 = jnp.sum(e * e, axis=0, keepdims=True)
    ca = 1.0 / jnp.maximum(jnp.sqrt(n2), EPS)
    sims = raw * inv_nb.reshape(16, 1)
    pooled = jnp.concatenate([
        jnp.max(sims[0:4], axis=0, keepdims=True),
        jnp.max(sims[4:9], axis=0, keepdims=True),
        sims[9:10],
        sims[10:11],
    ], axis=0) * ca
    w = wb_ref[0:A, :]
    bias = wb_ref[A:A + 1, :].reshape(A, 1)
    logits = jnp.dot(w, pooled, preferred_element_type=jnp.float32) + bias
    m = jnp.max(logits, axis=0, keepdims=True)
    es = jnp.exp(logits - m)
    out_ref[...] = es / jnp.sum(es, axis=0, keepdims=True)


_tc_decoder = pl.pallas_call(
    _tc_body,
    grid=(TC_B // TCBLK,),
    in_specs=[
        pl.BlockSpec((D, TCBLK), lambda i: (0, SC_B // TCBLK + i)),
        pl.BlockSpec((D, 16), lambda i: (0, 0)),
        pl.BlockSpec((2 * A, A), lambda i: (0, 0)),
    ],
    out_specs=pl.BlockSpec((A, TCBLK), lambda i: (0, i)),
    out_shape=jax.ShapeDtypeStruct((A, TC_B), jnp.float32),
)


def kernel(embedded_words, action_vectors, W, b):
    av = action_vectors.reshape(D * P).astype(jnp.float32)
    av_splat = jnp.broadcast_to(av[:, None], (D * P, LANES))
    wb = jnp.concatenate([W.reshape(A * A), b.reshape(A)]).astype(jnp.float32)
    wb_splat = jnp.broadcast_to(wb[:, None], (A * A + A, LANES))
    et4 = embedded_words.reshape(128, 128, 8, 8).transpose(2, 0, 3, 1)
    o_sc = _sc_decoder(et4, av_splat, wb_splat)
    et = embedded_words.T
    av_pad = jnp.zeros((D, 16), jnp.float32).at[:, :P].set(
        action_vectors.reshape(D, P).astype(jnp.float32))
    wb_tc = jnp.zeros((2 * A, A), jnp.float32).at[:A].set(W).at[A].set(b)
    o_tc = _tc_decoder(et, av_pad, wb_tc)
    o_tc2 = o_tc.reshape(A, TC_B // 128, 128).transpose(1, 0, 2).reshape(
        TC_B // 128 * A, 128)
    o2 = jnp.concatenate([o_sc, o_tc2], axis=0)
    return o2.reshape(B // 128, A, 128).transpose(0, 2, 1).reshape(B, A)
